# baseline (speedup 1.0000x reference)
import functools

import jax
import jax.numpy as jnp
from jax import lax
from jax.experimental import pallas as pl
from jax.experimental.pallas import tpu as pltpu
from jax.experimental.pallas import tpu_sc as plsc

N_NODES = 10000
N_EDGES = 320000
D = 128
NUM_REL = 237

_INFO = plsc.get_sparse_core_info()
NC = _INFO.num_cores
NS = _INFO.num_subcores
NW = NC * NS
LANES = 16
VPR = D // LANES

E_PER_W = N_EDGES // NW
CHUNK = 80
N_ITER = E_PER_W // CHUNK
N_PAIR = (N_ITER - 1) // 2
GROUPS = CHUNK // LANES


def _make_kernel():
    mesh = plsc.VectorSubcoreMesh(core_axis_name="c", subcore_axis_name="s")

    @functools.partial(
        pl.kernel,
        out_type=jax.ShapeDtypeStruct((N_EDGES,), jnp.float32),
        mesh=mesh,
        compiler_params=pltpu.CompilerParams(needs_layout_passes=False),
        scratch_types=[
            pltpu.VMEM((E_PER_W,), jnp.int32),
            pltpu.VMEM((CHUNK,), jnp.int32),
            pltpu.VMEM((CHUNK,), jnp.int32),
            pltpu.VMEM((CHUNK,), jnp.int32),
            pltpu.VMEM((CHUNK,), jnp.int32),
            pltpu.VMEM((NUM_REL, D), jnp.float32),
            pltpu.VMEM((CHUNK, D), jnp.float32),
            pltpu.VMEM((CHUNK, D), jnp.float32),
            pltpu.VMEM((CHUNK, D), jnp.float32),
            pltpu.VMEM((CHUNK, D), jnp.float32),
            pltpu.VMEM((E_PER_W,), jnp.float32),
            pltpu.VMEM((LANES * 17,), jnp.float32),
            pltpu.SemaphoreType.DMA,
            pltpu.SemaphoreType.DMA,
            pltpu.SemaphoreType.DMA,
            pltpu.SemaphoreType.DMA,
        ],
    )
    def k(x_hbm, h_hbm, t_hbm, et_hbm, rel_hbm, out_hbm,
          etidx, hi0, hi1, ti0, ti1, reltab, bh0, bh1, bt0, bt1, outbuf,
          tpose, semi0, semi1, semr0, semr1):
        wid = lax.axis_index("s") * NC + lax.axis_index("c")
        wbase = wid * E_PER_W

        lane = lax.iota(jnp.int32, 16)
        lane17 = lane * 17
        ibufs = ((hi0, ti0, semi0), (hi1, ti1, semi1))
        rbufs = ((bh0, bt0, semr0), (bh1, bt1, semr1))

        pltpu.sync_copy(et_hbm.at[pl.ds(wbase, E_PER_W)], etidx)
        pltpu.sync_copy(rel_hbm, reltab)

        def fire_idx(i, slot):
            hi, ti, sem = ibufs[slot]
            sl = pl.ds(wbase + i * CHUNK, CHUNK)
            pltpu.async_copy(h_hbm.at[sl], hi, sem)
            pltpu.async_copy(t_hbm.at[sl], ti, sem)

        def drain_idx(slot):
            hi, ti, sem = ibufs[slot]
            for b in (hi, ti):
                pltpu.make_async_copy(h_hbm.at[pl.ds(0, CHUNK)], b, sem).wait()

        def fire_rows(slot):
            hi, ti, _ = ibufs[slot]
            bh, bt, sem = rbufs[slot]
            pltpu.async_copy(x_hbm.at[hi], bh, sem)
            pltpu.async_copy(x_hbm.at[ti], bt, sem)

        def drain_rows(slot):
            hi, _, _ = ibufs[slot]
            bh, bt, sem = rbufs[slot]
            for b in (bh, bt):
                pltpu.make_async_copy(x_hbm.at[hi], b, sem).wait()

        def compute(i, slot):
            bh, bt, _ = rbufs[slot]
            obase = i * CHUNK
            for g in range(GROUPS):
                etv = etidx[pl.ds(obase + g * LANES, LANES)]
                for j in range(LANES):
                    e = g * LANES + j
                    rrow = reltab.at[etv[j]]
                    a = []
                    for kk in range(VPR):
                        sl = pl.ds(kk * LANES, LANES)
                        v = bh[e, sl] + rrow[sl] - bt[e, sl]
                        a.append(jnp.abs(v))
                    while len(a) > 1:
                        a = [a[m] + a[m + 1] for m in range(0, len(a), 2)]
                    plsc.store_scatter(tpose, [lane + (17 * j)], a[0])
                scores = None
                cols = []
                for c in range(LANES):
                    cols.append(plsc.load_gather(tpose, [lane17 + c]))
                while len(cols) > 1:
                    cols = [cols[m] + cols[m + 1] for m in range(0, len(cols), 2)]
                outbuf[pl.ds(obase + g * LANES, LANES)] = cols[0]

        def compute_dyn(i, slot):
            bh, bt, _ = rbufs[slot]
            obase = i * CHUNK

            def grp(g, _):
                etv = etidx[pl.ds(obase + g * LANES, LANES)]
                scores = jnp.zeros((LANES,), jnp.float32)
                for j in range(LANES):
                    rrow = reltab.at[etv[j]]
                    acc = None
                    for kk in range(VPR):
                        sl = pl.ds(kk * LANES, LANES)
                        v = bh[g * LANES + j, sl] + rrow[sl] - bt[g * LANES + j, sl]
                        a = jnp.abs(v)
                        acc = a if acc is None else acc + a
                    tot = jnp.sum(acc)
                    scores = jnp.where(lane == j, tot, scores)
                outbuf[pl.ds(obase + g * LANES, LANES)] = scores
                return ()

            lax.fori_loop(0, GROUPS, grp, (), unroll=False)

        fire_idx(0, 0)
        fire_idx(1, 1)
        drain_idx(0)
        fire_rows(0)

        def pair(p, _):
            i0 = 2 * p
            drain_rows(0)
            fire_idx(i0 + 2, 0)
            drain_idx(1)
            fire_rows(1)
            compute(i0, 0)
            drain_rows(1)
            @pl.when(i0 + 3 < N_ITER)
            def _():
                fire_idx(i0 + 3, 1)
            drain_idx(0)
            fire_rows(0)
            compute(i0 + 1, 1)
            return ()

        lax.fori_loop(0, N_PAIR, pair, (), unroll=False)

        drain_rows(0)
        compute_dyn(N_ITER - 1, 0)

        pltpu.sync_copy(outbuf, out_hbm.at[pl.ds(wbase, E_PER_W)])

    return k


_kernel_call = _make_kernel()


@jax.jit
def kernel(x, edge_index, edge_type, rel_embedding):
    h = edge_index[0]
    t = edge_index[1]
    return _kernel_call(x, h, t, edge_type, rel_embedding)


# speedup vs baseline: 1.7502x; 1.7502x over previous
import functools

import jax
import jax.numpy as jnp
from jax import lax
from jax.experimental import pallas as pl
from jax.experimental.pallas import tpu as pltpu
from jax.experimental.pallas import tpu_sc as plsc

N_NODES = 10000
N_EDGES = 320000
D = 128
NUM_REL = 237

_INFO = plsc.get_sparse_core_info()
NC = _INFO.num_cores
NS = _INFO.num_subcores
NW = NC * NS
LANES = 16
VPR = D // LANES

E_PER_W = N_EDGES // NW
CHUNK = 80
N_ITER = E_PER_W // CHUNK
GROUPS = CHUNK // LANES
NSLOT = 4
N_QUAD = 30


def _neg_body(x_ref, o_ref):
    o_ref[...] = -x_ref[...]


_neg_call = pl.pallas_call(
    _neg_body,
    out_shape=jax.ShapeDtypeStruct((N_NODES, D), jnp.float32),
)


def _make_kernel():
    mesh = plsc.VectorSubcoreMesh(core_axis_name="c", subcore_axis_name="s")

    scratch = []
    for _ in range(NSLOT):
        scratch.append(pltpu.VMEM((CHUNK,), jnp.int32))
        scratch.append(pltpu.VMEM((CHUNK,), jnp.int32))
        scratch.append(pltpu.VMEM((CHUNK,), jnp.int32))
        scratch.append(pltpu.VMEM((CHUNK, D), jnp.float32))
        scratch.append(pltpu.SemaphoreType.DMA)
        scratch.append(pltpu.SemaphoreType.DMA)
        scratch.append(pltpu.SemaphoreType.DMA)
        scratch.append(pltpu.SemaphoreType.DMA)
    scratch.append(pltpu.VMEM((E_PER_W,), jnp.float32))
    scratch.append(pltpu.VMEM((LANES * 17,), jnp.float32))

    @functools.partial(
        pl.kernel,
        out_type=jax.ShapeDtypeStruct((N_EDGES,), jnp.float32),
        mesh=mesh,
        compiler_params=pltpu.CompilerParams(needs_layout_passes=False),
        scratch_types=scratch,
    )
    def k(x_hbm, xneg_hbm, h_hbm, t_hbm, et_hbm, rel_hbm, out_hbm, *scr):
        slots = [scr[8 * s: 8 * s + 8] for s in range(NSLOT)]
        outbuf = scr[8 * NSLOT]
        tpose = scr[8 * NSLOT + 1]

        wid = lax.axis_index("s") * NC + lax.axis_index("c")
        wbase = wid * E_PER_W

        lane = lax.iota(jnp.int32, 16)
        lane17 = lane * 17

        def fire_idx(i, s):
            hi, ti, ei, _, semi, _, _, _ = slots[s]
            sl = pl.ds(wbase + i * CHUNK, CHUNK)
            pltpu.async_copy(h_hbm.at[sl], hi, semi)
            pltpu.async_copy(t_hbm.at[sl], ti, semi)
            pltpu.async_copy(et_hbm.at[sl], ei, semi)

        def drain_idx(s):
            hi, ti, ei, _, semi, _, _, _ = slots[s]
            for b in (hi, ti, ei):
                pltpu.make_async_copy(h_hbm.at[pl.ds(0, CHUNK)], b, semi).wait()

        def fire_h(s):
            hi, _, _, buf, _, semh, _, _ = slots[s]
            pltpu.async_copy(x_hbm.at[hi], buf, semh)

        def drain_h(s):
            hi, _, _, buf, _, semh, _, _ = slots[s]
            pltpu.make_async_copy(x_hbm.at[hi], buf, semh).wait()

        def fire_r(s):
            _, _, ei, buf, _, _, semr, _ = slots[s]
            pltpu.async_copy(rel_hbm.at[ei], buf, semr, add=True)

        def drain_r(s):
            _, _, ei, buf, _, _, semr, _ = slots[s]
            pltpu.make_async_copy(rel_hbm.at[ei], buf, semr).wait()

        def fire_t(s):
            _, ti, _, buf, _, _, _, semt = slots[s]
            pltpu.async_copy(xneg_hbm.at[ti], buf, semt, add=True)

        def drain_t(s):
            _, ti, _, buf, _, _, _, semt = slots[s]
            pltpu.make_async_copy(xneg_hbm.at[ti], buf, semt).wait()

        def compute(i, s):
            buf = slots[s][3]
            obase = i * CHUNK
            for g in range(GROUPS):
                for j in range(LANES):
                    e = g * LANES + j
                    a = []
                    for kk in range(VPR):
                        v = buf[e, pl.ds(kk * LANES, LANES)]
                        a.append(jnp.abs(v))
                    while len(a) > 1:
                        a = [a[m] + a[m + 1] for m in range(0, len(a), 2)]
                    plsc.store_scatter(tpose, [lane + (17 * j)], a[0])
                cols = []
                for c in range(LANES):
                    cols.append(plsc.load_gather(tpose, [lane17 + c]))
                while len(cols) > 1:
                    cols = [cols[m] + cols[m + 1] for m in range(0, len(cols), 2)]
                outbuf[pl.ds(obase + g * LANES, LANES)] = cols[0]

        def compute_dyn(i, s):
            buf = slots[s][3]
            obase = i * CHUNK

            def grp(g, _):
                for j in range(LANES):
                    a = []
                    for kk in range(VPR):
                        v = buf[g * LANES + j, pl.ds(kk * LANES, LANES)]
                        a.append(jnp.abs(v))
                    while len(a) > 1:
                        a = [a[m] + a[m + 1] for m in range(0, len(a), 2)]
                    plsc.store_scatter(tpose, [lane + (17 * j)], a[0])
                cols = []
                for c in range(LANES):
                    cols.append(plsc.load_gather(tpose, [lane17 + c]))
                while len(cols) > 1:
                    cols = [cols[m] + cols[m + 1] for m in range(0, len(cols), 2)]
                outbuf[pl.ds(obase + g * LANES, LANES)] = cols[0]
                return ()

            lax.fori_loop(0, GROUPS, grp, (), unroll=False)

        for s in range(NSLOT):
            fire_idx(s, s)
        drain_idx(0)
        fire_h(0)
        drain_idx(1)
        fire_h(1)
        drain_h(0)
        fire_r(0)
        drain_idx(2)
        fire_h(2)
        drain_r(0)
        fire_t(0)
        drain_h(1)
        fire_r(1)

        def quad(q, _):
            i0 = 4 * q
            for u in range(NSLOT):
                i = i0 + u
                drain_t(u)
                fire_idx(i + 4, u)
                drain_idx((u + 3) % 4)
                fire_h((u + 3) % 4)
                drain_h((u + 2) % 4)
                fire_r((u + 2) % 4)
                drain_r((u + 1) % 4)
                fire_t((u + 1) % 4)
                compute(i, u)
            return ()

        lax.fori_loop(0, N_QUAD, quad, (), unroll=False)

        for i in range(4 * N_QUAD, N_ITER):
            u = i % 4
            drain_t(u)
            if i + 4 < N_ITER:
                fire_idx(i + 4, u)
            if i + 3 < N_ITER:
                drain_idx((u + 3) % 4)
                fire_h((u + 3) % 4)
            if i + 2 < N_ITER:
                drain_h((u + 2) % 4)
                fire_r((u + 2) % 4)
            if i + 1 < N_ITER:
                drain_r((u + 1) % 4)
                fire_t((u + 1) % 4)
            compute_dyn(i, u)

        pltpu.sync_copy(outbuf, out_hbm.at[pl.ds(wbase, E_PER_W)])

    return k


_kernel_call = _make_kernel()


@jax.jit
def kernel(x, edge_index, edge_type, rel_embedding):
    xneg = _neg_call(x)
    h = edge_index[0]
    t = edge_index[1]
    return _kernel_call(x, xneg, h, t, edge_type, rel_embedding)
